# baseline (speedup 1.0000x reference)
import functools

import jax
import jax.numpy as jnp
from jax import lax
from jax.experimental import pallas as pl
from jax.experimental.pallas import tpu as pltpu
from jax.experimental.pallas import tpu_sc as plsc

_N = 10000
_E = 320000
_D = 128
_B = 128

_R = 1024
_NPAD = 10240
_NB = _NPAD // _R

_NW = 32
_CPE = 128
_NCHUNK = 80
_EPAD = _NW * _NCHUNK * _CPE
_RPW = _NPAD // 16


def _tc_stage1_body(x_ref, w_ref, a2_ref, hlo_ref, hhi_ref, sa_ref, gm_ref):
    i = pl.program_id(0)
    h = jnp.dot(x_ref[...], w_ref[...], preferred_element_type=jnp.float32)
    hlo_ref[...] = h[:, : _D // 2]
    hhi_ref[...] = h[:, _D // 2:]
    sa = jnp.dot(h, a2_ref[...], preferred_element_type=jnp.float32)
    sa_ref[...] = sa
    m = jnp.max(sa, axis=0, keepdims=True)

    @pl.when(i == 0)
    def _():
        gm_ref[...] = m

    @pl.when(i > 0)
    def _():
        gm_ref[...] = jnp.maximum(gm_ref[...], m)


def _tc_stage1(xp, W, a2):
    return pl.pallas_call(
        _tc_stage1_body,
        grid=(_NB,),
        in_specs=[
            pl.BlockSpec((_R, _D), lambda i: (i, 0)),
            pl.BlockSpec((_D, _D), lambda i: (0, 0)),
            pl.BlockSpec((_D, 2), lambda i: (0, 0)),
        ],
        out_specs=[
            pl.BlockSpec((_R, _D // 2), lambda i: (i, 0)),
            pl.BlockSpec((_R, _D // 2), lambda i: (i, 0)),
            pl.BlockSpec((_R, 2), lambda i: (i, 0)),
            pl.BlockSpec((1, 2), lambda i: (0, 0)),
        ],
        out_shape=[
            jax.ShapeDtypeStruct((_NPAD, _D // 2), jnp.float32),
            jax.ShapeDtypeStruct((_NPAD, _D // 2), jnp.float32),
            jax.ShapeDtypeStruct((_NPAD, 2), jnp.float32),
            jax.ShapeDtypeStruct((1, 2), jnp.float32),
        ],
    )(xp, W, a2)


def _combine(o_ref, d_ref, b_ref):
    o_lo = o_ref[0, 0] + o_ref[1, 0]
    o_hi = o_ref[0, 1] + o_ref[1, 1]
    o = jnp.concatenate([o_lo, o_hi], axis=1)
    den = d_ref[0] + d_ref[1]
    z = o / (den + 1e-16) + b_ref[...]
    return jnp.where(z >= 0.0, z, 0.01 * z)


def _tc_stage2_body(o_ref, d_ref, b_ref, w_ref, a2_ref,
                    hlo_ref, hhi_ref, sa_ref, gm_ref):
    i = pl.program_id(0)
    z = _combine(o_ref, d_ref, b_ref)
    h = jnp.dot(z, w_ref[...], preferred_element_type=jnp.float32)
    hlo_ref[...] = h[:, : _D // 2]
    hhi_ref[...] = h[:, _D // 2:]
    sa = jnp.dot(h, a2_ref[...], preferred_element_type=jnp.float32)
    sa_ref[...] = sa
    m = jnp.max(sa, axis=0, keepdims=True)

    @pl.when(i == 0)
    def _():
        gm_ref[...] = m

    @pl.when(i > 0)
    def _():
        gm_ref[...] = jnp.maximum(gm_ref[...], m)


def _tc_stage2(o_parts, d_parts3, b, W, a2):
    return pl.pallas_call(
        _tc_stage2_body,
        grid=(_NB,),
        in_specs=[
            pl.BlockSpec((2, 2, _R, _D // 2), lambda i: (0, 0, i, 0)),
            pl.BlockSpec((2, _R, 1), lambda i: (0, i, 0)),
            pl.BlockSpec((1, _D), lambda i: (0, 0)),
            pl.BlockSpec((_D, _D), lambda i: (0, 0)),
            pl.BlockSpec((_D, 2), lambda i: (0, 0)),
        ],
        out_specs=[
            pl.BlockSpec((_R, _D // 2), lambda i: (i, 0)),
            pl.BlockSpec((_R, _D // 2), lambda i: (i, 0)),
            pl.BlockSpec((_R, 2), lambda i: (i, 0)),
            pl.BlockSpec((1, 2), lambda i: (0, 0)),
        ],
        out_shape=[
            jax.ShapeDtypeStruct((_NPAD, _D // 2), jnp.float32),
            jax.ShapeDtypeStruct((_NPAD, _D // 2), jnp.float32),
            jax.ShapeDtypeStruct((_NPAD, 2), jnp.float32),
            jax.ShapeDtypeStruct((1, 2), jnp.float32),
        ],
    )(o_parts, d_parts3, b, W, a2)


def _tc_final_body(o_ref, d_ref, b_ref, gf_ref, gb_ref, gy_ref, gm_ref):
    i = pl.program_id(0)
    z = _combine(o_ref, d_ref, b_ref)
    gy = jnp.dot(z, gf_ref[...], preferred_element_type=jnp.float32)
    gy = gy + gb_ref[...]
    gy_ref[...] = gy
    m = jnp.max(gy[:, 0:1], axis=0, keepdims=True)

    @pl.when(i == 0)
    def _():
        gm_ref[...] = m

    @pl.when(i > 0)
    def _():
        gm_ref[...] = jnp.maximum(gm_ref[...], m)


def _tc_final(o_parts, d_parts3, b, gf, gb2):
    return pl.pallas_call(
        _tc_final_body,
        grid=(_NB,),
        in_specs=[
            pl.BlockSpec((2, 2, _R, _D // 2), lambda i: (0, 0, i, 0)),
            pl.BlockSpec((2, _R, 1), lambda i: (0, i, 0)),
            pl.BlockSpec((1, _D), lambda i: (0, 0)),
            pl.BlockSpec((_D, 2), lambda i: (0, 0)),
            pl.BlockSpec((1, 2), lambda i: (0, 0)),
        ],
        out_specs=[
            pl.BlockSpec((_R, 2), lambda i: (i, 0)),
            pl.BlockSpec((1, 1), lambda i: (0, 0)),
        ],
        out_shape=[
            jax.ShapeDtypeStruct((_NPAD, 2), jnp.float32),
            jax.ShapeDtypeStruct((1, 1), jnp.float32),
        ],
    )(o_parts, d_parts3, b, gf, gb2)


def _tc_pool_body(gy_ref, batch_ref, gm_ref, fb_ref, acc_ref, out_ref):
    i = pl.program_id(0)
    g = gy_ref[:, 0:1]
    y = gy_ref[:, 1:2]
    p = jnp.exp(g - gm_ref[...])
    py = p * y
    b_row = batch_ref[0]
    ids = lax.broadcasted_iota(jnp.int32, (_B, 1), 0)
    oh = jnp.where(b_row == ids, 1.0, 0.0)
    contrib = jnp.dot(oh, jnp.concatenate([py, p], axis=1),
                      preferred_element_type=jnp.float32)

    @pl.when(i == 0)
    def _():
        acc_ref[...] = contrib

    @pl.when(i > 0)
    def _():
        acc_ref[...] = acc_ref[...] + contrib

    @pl.when(i == _NB - 1)
    def _():
        a = acc_ref[...]
        out_ref[...] = a[:, 0:1] / (a[:, 1:2] + 1e-16) + fb_ref[...]


def _tc_pool(gy, batch3, gm, fb):
    return pl.pallas_call(
        _tc_pool_body,
        grid=(_NB,),
        in_specs=[
            pl.BlockSpec((_R, 2), lambda i: (i, 0)),
            pl.BlockSpec((1, 1, _R), lambda i: (i, 0, 0)),
            pl.BlockSpec((1, 1), lambda i: (0, 0)),
            pl.BlockSpec((1, 1), lambda i: (0, 0)),
        ],
        out_specs=[
            pl.BlockSpec((_B, 2), lambda i: (0, 0)),
            pl.BlockSpec((_B, 1), lambda i: (0, 0)),
        ],
        out_shape=[
            jax.ShapeDtypeStruct((_B, 2), jnp.float32),
            jax.ShapeDtypeStruct((_B, 1), jnp.float32),
        ],
    )(gy, batch3, gm, fb)


def _sc_edge_body(src_hbm, dst_hbm, asrc_hbm, adst_hbm, gm_hbm,
                  hlo_hbm, hhi_hbm,
                  out_hbm, den_hbm,
                  asrc_v, adst_v, gm_v, src_all, dst_all, p_all,
                  rows0, rows1, zrow, zden, out_acc, den_acc,
                  gsem0, gsem1, ssem0, ssem1):
    c = lax.axis_index("c")
    s = lax.axis_index("s")
    hd = _D // 2

    pltpu.sync_copy(asrc_hbm, asrc_v)
    pltpu.sync_copy(adst_hbm, adst_v)
    pltpu.sync_copy(gm_hbm, gm_v)
    w = c * 16 + s
    pltpu.sync_copy(src_hbm.at[w], src_all)
    pltpu.sync_copy(dst_hbm.at[w], dst_all)

    zero16 = jnp.zeros((16,), jnp.float32)

    def _zrow_body(i, carry):
        for k in range(hd // 16):
            zrow[i, pl.ds(k * 16, 16)] = zero16
        return carry

    lax.fori_loop(0, _CPE, _zrow_body, 0)

    def _zden_body(i, carry):
        zden[pl.ds(i * 16, 16)] = zero16
        return carry

    lax.fori_loop(0, _RPW // 16, _zden_body, 0)

    r0 = s * _RPW
    gm = gm_v[...]
    lane = lax.iota(jnp.int32, 16)

    def _zero_my_out_slice():
        for k in range(_RPW // _CPE):
            pltpu.sync_copy(zrow, out_acc.at[pl.ds(r0 + k * _CPE, _CPE), :])

    _zero_my_out_slice()
    pltpu.sync_copy(zden, den_acc.at[pl.ds(r0, _RPW)])
    plsc.subcore_barrier()

    def _score_chunk(j, carry):
        base = (w * _NCHUNK + j) * _CPE
        for v in range(_CPE // 16):
            sv = src_all[j, pl.ds(v * 16, 16)]
            dv = dst_all[j, pl.ds(v * 16, 16)]
            va = plsc.load_gather(asrc_v, [sv])
            vb = plsc.load_gather(adst_v, [dv])
            t = va + vb
            t = jnp.where(t >= 0.0, t, t * 0.2)
            p = jnp.exp(t - gm)
            eid = base + v * 16 + lane
            p = jnp.where(eid < _E, p, 0.0)
            p_all[pl.ds(j * _CPE + v * 16, 16)] = p
        pltpu.sync_copy(p_all.at[pl.ds(j * _CPE, _CPE)],
                        den_acc.at[dst_all.at[j]], add=True)
        return carry

    lax.fori_loop(0, _NCHUNK, _score_chunk, 0)

    def _scale(rws, j):
        def _grp(g, rcarry):
            pv16 = p_all[pl.ds(j * _CPE + g * 16, 16)]
            for l in range(16):
                pv = pv16[l]
                r = g * 16 + l
                for k in range(hd // 16):
                    rws[r, pl.ds(k * 16, 16)] = rws[r, pl.ds(k * 16, 16)] * pv
            return rcarry

        lax.fori_loop(0, _CPE // 16, _grp, 0)

    for half in range(2):
        h_hbm = hlo_hbm if half == 0 else hhi_hbm
        if half == 1:
            _zero_my_out_slice()
            plsc.subcore_barrier()
        pltpu.async_copy(h_hbm.at[src_all.at[0]], rows0, gsem0)

        def _pair(i, carry):
            a = 2 * i
            b = 2 * i + 1

            @pl.when(i > 0)
            def _():
                pltpu.make_async_copy(rows1, out_acc.at[dst_all.at[b - 2]],
                                      ssem1).wait()

            pltpu.async_copy(h_hbm.at[src_all.at[b]], rows1, gsem1)
            pltpu.make_async_copy(h_hbm.at[src_all.at[a]], rows0, gsem0).wait()
            _scale(rows0, a)
            pltpu.async_copy(rows0, out_acc.at[dst_all.at[a]], ssem0, add=True)
            pltpu.make_async_copy(h_hbm.at[src_all.at[b]], rows1, gsem1).wait()
            _scale(rows1, b)
            pltpu.async_copy(rows1, out_acc.at[dst_all.at[b]], ssem1, add=True)
            pltpu.make_async_copy(rows0, out_acc.at[dst_all.at[a]],
                                  ssem0).wait()

            @pl.when(i < _NCHUNK // 2 - 1)
            def _():
                pltpu.async_copy(h_hbm.at[src_all.at[a + 2]], rows0, gsem0)

            return carry

        lax.fori_loop(0, _NCHUNK // 2, _pair, 0)
        pltpu.make_async_copy(rows1, out_acc.at[dst_all.at[_NCHUNK - 1]],
                              ssem1).wait()
        plsc.subcore_barrier()

        for k in range(_RPW // _CPE):
            pltpu.sync_copy(out_acc.at[pl.ds(r0 + k * _CPE, _CPE), :],
                            out_hbm.at[c, half, pl.ds(r0 + k * _CPE, _CPE), :])
        if half == 0:
            pltpu.sync_copy(den_acc.at[pl.ds(r0, _RPW)],
                            den_hbm.at[c, pl.ds(r0, _RPW)])


@functools.cache
def _get_sc_edge():
    return pl.kernel(
        _sc_edge_body,
        out_type=(
            jax.ShapeDtypeStruct((2, 2, _NPAD, _D // 2), jnp.float32),
            jax.ShapeDtypeStruct((2, _NPAD), jnp.float32),
        ),
        mesh=plsc.VectorSubcoreMesh(core_axis_name="c", subcore_axis_name="s"),
        compiler_params=pltpu.CompilerParams(needs_layout_passes=False,
                                             use_tc_tiling_on_sc=False),
        scratch_types=[
            pltpu.VMEM((_NPAD,), jnp.float32),
            pltpu.VMEM((_NPAD,), jnp.float32),
            pltpu.VMEM((16,), jnp.float32),
            pltpu.VMEM((_NCHUNK, _CPE), jnp.int32),
            pltpu.VMEM((_NCHUNK, _CPE), jnp.int32),
            pltpu.VMEM((_NCHUNK * _CPE,), jnp.float32),
            pltpu.VMEM((_CPE, _D // 2), jnp.float32),
            pltpu.VMEM((_CPE, _D // 2), jnp.float32),
            pltpu.VMEM((_CPE, _D // 2), jnp.float32),
            pltpu.VMEM((_RPW,), jnp.float32),
            pltpu.VMEM_SHARED((_NPAD, _D // 2), jnp.float32),
            pltpu.VMEM_SHARED((_NPAD,), jnp.float32),
            pltpu.SemaphoreType.DMA,
            pltpu.SemaphoreType.DMA,
            pltpu.SemaphoreType.DMA,
            pltpu.SemaphoreType.DMA,
        ],
    )


def kernel(x, edge_index, batch, W1, a_src1, a_dst1, b1,
           W2, a_src2, a_dst2, b2, gate_W, gate_b, fin_W, fin_b):
    f32 = jnp.float32
    xp = jnp.pad(x, ((0, _NPAD - _N), (0, 0)))
    srcp = jnp.pad(edge_index[0], (0, _EPAD - _E)).reshape(_NW, _NCHUNK, _CPE)
    dstp = jnp.pad(edge_index[1], (0, _EPAD - _E)).reshape(_NW, _NCHUNK, _CPE)
    batch3 = jnp.pad(batch, (0, _NPAD - _N),
                     constant_values=_B).reshape(_NB, 1, _R)
    a21 = jnp.stack([a_src1, a_dst1], axis=1)
    a22 = jnp.stack([a_src2, a_dst2], axis=1)
    b1r = b1.reshape(1, _D)
    b2r = b2.reshape(1, _D)
    gf = jnp.concatenate([gate_W, fin_W], axis=1)
    gb2 = jnp.concatenate([gate_b, jnp.zeros((1,), f32)]).reshape(1, 2)
    fbr = fin_b.reshape(1, 1)

    sc_edge = _get_sc_edge()
    h1lo, h1hi, sa1, gm1 = _tc_stage1(xp, W1, a21)
    gv1 = jnp.full((16,), gm1[0, 0] + gm1[0, 1], dtype=f32)
    o1, d1 = sc_edge(srcp, dstp, sa1[:, 0], sa1[:, 1], gv1, h1lo, h1hi)

    h2lo, h2hi, sa2, gm2 = _tc_stage2(o1, d1[:, :, None], b1r, W2, a22)
    gv2 = jnp.full((16,), gm2[0, 0] + gm2[0, 1], dtype=f32)
    o2, d2 = sc_edge(srcp, dstp, sa2[:, 0], sa2[:, 1], gv2, h2lo, h2hi)

    gy, gm3 = _tc_final(o2, d2[:, :, None], b2r, gf, gb2)
    _, outf = _tc_pool(gy, batch3, gm3, fbr)
    return outf[:, 0]


# speedup vs baseline: 1.2264x; 1.2264x over previous
import functools

import jax
import jax.numpy as jnp
from jax import lax
from jax.experimental import pallas as pl
from jax.experimental.pallas import tpu as pltpu
from jax.experimental.pallas import tpu_sc as plsc

_N = 10000
_E = 320000
_D = 128
_B = 128

_R = 1024
_NPAD = 10240
_NB = _NPAD // _R

_NW = 32
_CPE = 128
_NCHUNK = 80
_EPAD = _NW * _NCHUNK * _CPE
_RPW = _NPAD // 16


def _tc_stage1_body(x_ref, w_ref, a2_ref, hlo_ref, hhi_ref, sa_ref, gm_ref):
    i = pl.program_id(0)
    h = jnp.dot(x_ref[...], w_ref[...], preferred_element_type=jnp.float32)
    hlo_ref[...] = h[:, : _D // 2]
    hhi_ref[...] = h[:, _D // 2:]
    sa = jnp.dot(h, a2_ref[...], preferred_element_type=jnp.float32)
    sa_ref[...] = sa
    m = jnp.max(sa, axis=0, keepdims=True)

    @pl.when(i == 0)
    def _():
        gm_ref[...] = m

    @pl.when(i > 0)
    def _():
        gm_ref[...] = jnp.maximum(gm_ref[...], m)


def _tc_stage1(xp, W, a2):
    return pl.pallas_call(
        _tc_stage1_body,
        grid=(_NB,),
        in_specs=[
            pl.BlockSpec((_R, _D), lambda i: (i, 0)),
            pl.BlockSpec((_D, _D), lambda i: (0, 0)),
            pl.BlockSpec((_D, 2), lambda i: (0, 0)),
        ],
        out_specs=[
            pl.BlockSpec((_R, _D // 2), lambda i: (i, 0)),
            pl.BlockSpec((_R, _D // 2), lambda i: (i, 0)),
            pl.BlockSpec((_R, 2), lambda i: (i, 0)),
            pl.BlockSpec((1, 2), lambda i: (0, 0)),
        ],
        out_shape=[
            jax.ShapeDtypeStruct((_NPAD, _D // 2), jnp.float32),
            jax.ShapeDtypeStruct((_NPAD, _D // 2), jnp.float32),
            jax.ShapeDtypeStruct((_NPAD, 2), jnp.float32),
            jax.ShapeDtypeStruct((1, 2), jnp.float32),
        ],
    )(xp, W, a2)


def _combine(o_ref, d_ref, b_ref):
    o_lo = o_ref[0, 0] + o_ref[1, 0]
    o_hi = o_ref[0, 1] + o_ref[1, 1]
    o = jnp.concatenate([o_lo, o_hi], axis=1)
    den = d_ref[0] + d_ref[1]
    z = o / (den + 1e-16) + b_ref[...]
    return jnp.where(z >= 0.0, z, 0.01 * z)


def _tc_stage2_body(o_ref, d_ref, b_ref, w_ref, a2_ref,
                    hlo_ref, hhi_ref, sa_ref, gm_ref):
    i = pl.program_id(0)
    z = _combine(o_ref, d_ref, b_ref)
    h = jnp.dot(z, w_ref[...], preferred_element_type=jnp.float32)
    hlo_ref[...] = h[:, : _D // 2]
    hhi_ref[...] = h[:, _D // 2:]
    sa = jnp.dot(h, a2_ref[...], preferred_element_type=jnp.float32)
    sa_ref[...] = sa
    m = jnp.max(sa, axis=0, keepdims=True)

    @pl.when(i == 0)
    def _():
        gm_ref[...] = m

    @pl.when(i > 0)
    def _():
        gm_ref[...] = jnp.maximum(gm_ref[...], m)


def _tc_stage2(o_parts, d_parts3, b, W, a2):
    return pl.pallas_call(
        _tc_stage2_body,
        grid=(_NB,),
        in_specs=[
            pl.BlockSpec((2, 2, _R, _D // 2), lambda i: (0, 0, i, 0)),
            pl.BlockSpec((2, _R, 1), lambda i: (0, i, 0)),
            pl.BlockSpec((1, _D), lambda i: (0, 0)),
            pl.BlockSpec((_D, _D), lambda i: (0, 0)),
            pl.BlockSpec((_D, 2), lambda i: (0, 0)),
        ],
        out_specs=[
            pl.BlockSpec((_R, _D // 2), lambda i: (i, 0)),
            pl.BlockSpec((_R, _D // 2), lambda i: (i, 0)),
            pl.BlockSpec((_R, 2), lambda i: (i, 0)),
            pl.BlockSpec((1, 2), lambda i: (0, 0)),
        ],
        out_shape=[
            jax.ShapeDtypeStruct((_NPAD, _D // 2), jnp.float32),
            jax.ShapeDtypeStruct((_NPAD, _D // 2), jnp.float32),
            jax.ShapeDtypeStruct((_NPAD, 2), jnp.float32),
            jax.ShapeDtypeStruct((1, 2), jnp.float32),
        ],
    )(o_parts, d_parts3, b, W, a2)


def _tc_final_body(o_ref, d_ref, b_ref, gf_ref, gb_ref, gy_ref, gm_ref):
    i = pl.program_id(0)
    z = _combine(o_ref, d_ref, b_ref)
    gy = jnp.dot(z, gf_ref[...], preferred_element_type=jnp.float32)
    gy = gy + gb_ref[...]
    gy_ref[...] = gy
    m = jnp.max(gy[:, 0:1], axis=0, keepdims=True)

    @pl.when(i == 0)
    def _():
        gm_ref[...] = m

    @pl.when(i > 0)
    def _():
        gm_ref[...] = jnp.maximum(gm_ref[...], m)


def _tc_final(o_parts, d_parts3, b, gf, gb2):
    return pl.pallas_call(
        _tc_final_body,
        grid=(_NB,),
        in_specs=[
            pl.BlockSpec((2, 2, _R, _D // 2), lambda i: (0, 0, i, 0)),
            pl.BlockSpec((2, _R, 1), lambda i: (0, i, 0)),
            pl.BlockSpec((1, _D), lambda i: (0, 0)),
            pl.BlockSpec((_D, 2), lambda i: (0, 0)),
            pl.BlockSpec((1, 2), lambda i: (0, 0)),
        ],
        out_specs=[
            pl.BlockSpec((_R, 2), lambda i: (i, 0)),
            pl.BlockSpec((1, 1), lambda i: (0, 0)),
        ],
        out_shape=[
            jax.ShapeDtypeStruct((_NPAD, 2), jnp.float32),
            jax.ShapeDtypeStruct((1, 1), jnp.float32),
        ],
    )(o_parts, d_parts3, b, gf, gb2)


def _tc_pool_body(gy_ref, batch_ref, gm_ref, fb_ref, acc_ref, out_ref):
    i = pl.program_id(0)
    g = gy_ref[:, 0:1]
    y = gy_ref[:, 1:2]
    p = jnp.exp(g - gm_ref[...])
    py = p * y
    b_row = batch_ref[0]
    ids = lax.broadcasted_iota(jnp.int32, (_B, 1), 0)
    oh = jnp.where(b_row == ids, 1.0, 0.0)
    contrib = jnp.dot(oh, jnp.concatenate([py, p], axis=1),
                      preferred_element_type=jnp.float32)

    @pl.when(i == 0)
    def _():
        acc_ref[...] = contrib

    @pl.when(i > 0)
    def _():
        acc_ref[...] = acc_ref[...] + contrib

    @pl.when(i == _NB - 1)
    def _():
        a = acc_ref[...]
        out_ref[...] = a[:, 0:1] / (a[:, 1:2] + 1e-16) + fb_ref[...]


def _tc_pool(gy, batch3, gm, fb):
    return pl.pallas_call(
        _tc_pool_body,
        grid=(_NB,),
        in_specs=[
            pl.BlockSpec((_R, 2), lambda i: (i, 0)),
            pl.BlockSpec((1, 1, _R), lambda i: (i, 0, 0)),
            pl.BlockSpec((1, 1), lambda i: (0, 0)),
            pl.BlockSpec((1, 1), lambda i: (0, 0)),
        ],
        out_specs=[
            pl.BlockSpec((_B, 2), lambda i: (0, 0)),
            pl.BlockSpec((_B, 1), lambda i: (0, 0)),
        ],
        out_shape=[
            jax.ShapeDtypeStruct((_B, 2), jnp.float32),
            jax.ShapeDtypeStruct((_B, 1), jnp.float32),
        ],
    )(gy, batch3, gm, fb)


def _sc_edge_body(src_hbm, dst_hbm, asrc_hbm, adst_hbm, gm_hbm,
                  hlo_hbm, hhi_hbm,
                  out_hbm, den_hbm,
                  asrc_v, adst_v, gm_v, src_all, dst_all, p_all,
                  rows0, rows1, zrow, zden, out_acc, den_acc,
                  gsem0, gsem1, ssem0, ssem1):
    c = lax.axis_index("c")
    s = lax.axis_index("s")
    hd = _D // 2

    pltpu.sync_copy(asrc_hbm, asrc_v)
    pltpu.sync_copy(adst_hbm, adst_v)
    pltpu.sync_copy(gm_hbm, gm_v)
    w = c * 16 + s
    pltpu.sync_copy(src_hbm.at[w], src_all)
    pltpu.sync_copy(dst_hbm.at[w], dst_all)

    zero16 = jnp.zeros((16,), jnp.float32)

    def _zrow_body(i, carry):
        for k in range(hd // 16):
            zrow[i, pl.ds(k * 16, 16)] = zero16
        return carry

    lax.fori_loop(0, _CPE, _zrow_body, 0)

    def _zden_body(i, carry):
        zden[pl.ds(i * 16, 16)] = zero16
        return carry

    lax.fori_loop(0, _RPW // 16, _zden_body, 0)

    r0 = s * _RPW
    gm = gm_v[...]
    lane = lax.iota(jnp.int32, 16)

    def _zero_my_out_slice():
        for k in range(_RPW // _CPE):
            pltpu.sync_copy(zrow, out_acc.at[pl.ds(r0 + k * _CPE, _CPE), :])

    _zero_my_out_slice()
    pltpu.sync_copy(zden, den_acc.at[pl.ds(r0, _RPW)])
    plsc.subcore_barrier()

    def _score_chunk(j, carry):
        base = (w * _NCHUNK + j) * _CPE
        for v in range(_CPE // 16):
            sv = src_all[j, pl.ds(v * 16, 16)]
            dv = dst_all[j, pl.ds(v * 16, 16)]
            va = plsc.load_gather(asrc_v, [sv])
            vb = plsc.load_gather(adst_v, [dv])
            t = va + vb
            t = jnp.where(t >= 0.0, t, t * 0.2)
            p = jnp.exp(t - gm)
            eid = base + v * 16 + lane
            p = jnp.where(eid < _E, p, 0.0)
            p_all[pl.ds(j * _CPE + v * 16, 16)] = p
        pltpu.sync_copy(p_all.at[pl.ds(j * _CPE, _CPE)],
                        den_acc.at[dst_all.at[j]], add=True)
        return carry

    lax.fori_loop(0, _NCHUNK, _score_chunk, 0)

    def _scale(rws, j):
        def _grp(g, rcarry):
            pv16 = p_all[pl.ds(j * _CPE + g * 16, 16)]
            for l in range(16):
                pv = pv16[l]
                r = g * 16 + l
                for k in range(hd // 16):
                    rws[r, pl.ds(k * 16, 16)] = rws[r, pl.ds(k * 16, 16)] * pv
            return rcarry

        lax.fori_loop(0, _CPE // 16, _grp, 0)

    for half in range(2):
        h_hbm = hlo_hbm if half == 0 else hhi_hbm
        if half == 1:
            _zero_my_out_slice()
            plsc.subcore_barrier()
        pltpu.async_copy(h_hbm.at[src_all.at[0]], rows0, gsem0)

        def _pair(i, carry):
            a = 2 * i
            b = 2 * i + 1
            pltpu.async_copy(h_hbm.at[src_all.at[b]], rows1, gsem1)
            pltpu.make_async_copy(h_hbm.at[src_all.at[a]], rows0, gsem0).wait()
            _scale(rows0, a)
            pltpu.sync_copy(rows0, out_acc.at[dst_all.at[a]], add=True)
            a2 = jnp.minimum(a + 2, _NCHUNK - 1)
            pltpu.async_copy(h_hbm.at[src_all.at[a2]], rows0, gsem0)
            pltpu.make_async_copy(h_hbm.at[src_all.at[b]], rows1, gsem1).wait()
            _scale(rows1, b)
            pltpu.sync_copy(rows1, out_acc.at[dst_all.at[b]], add=True)
            return carry

        lax.fori_loop(0, _NCHUNK // 2, _pair, 0)
        pltpu.make_async_copy(h_hbm.at[src_all.at[_NCHUNK - 1]], rows0,
                              gsem0).wait()
        plsc.subcore_barrier()

        for k in range(_RPW // _CPE):
            pltpu.sync_copy(out_acc.at[pl.ds(r0 + k * _CPE, _CPE), :],
                            out_hbm.at[c, half, pl.ds(r0 + k * _CPE, _CPE), :])
        if half == 0:
            pltpu.sync_copy(den_acc.at[pl.ds(r0, _RPW)],
                            den_hbm.at[c, pl.ds(r0, _RPW)])


@functools.cache
def _get_sc_edge():
    return pl.kernel(
        _sc_edge_body,
        out_type=(
            jax.ShapeDtypeStruct((2, 2, _NPAD, _D // 2), jnp.float32),
            jax.ShapeDtypeStruct((2, _NPAD), jnp.float32),
        ),
        mesh=plsc.VectorSubcoreMesh(core_axis_name="c", subcore_axis_name="s"),
        compiler_params=pltpu.CompilerParams(needs_layout_passes=False,
                                             use_tc_tiling_on_sc=False),
        scratch_types=[
            pltpu.VMEM((_NPAD,), jnp.float32),
            pltpu.VMEM((_NPAD,), jnp.float32),
            pltpu.VMEM((16,), jnp.float32),
            pltpu.VMEM((_NCHUNK, _CPE), jnp.int32),
            pltpu.VMEM((_NCHUNK, _CPE), jnp.int32),
            pltpu.VMEM((_NCHUNK * _CPE,), jnp.float32),
            pltpu.VMEM((_CPE, _D // 2), jnp.float32),
            pltpu.VMEM((_CPE, _D // 2), jnp.float32),
            pltpu.VMEM((_CPE, _D // 2), jnp.float32),
            pltpu.VMEM((_RPW,), jnp.float32),
            pltpu.VMEM_SHARED((_NPAD, _D // 2), jnp.float32),
            pltpu.VMEM_SHARED((_NPAD,), jnp.float32),
            pltpu.SemaphoreType.DMA,
            pltpu.SemaphoreType.DMA,
            pltpu.SemaphoreType.DMA,
            pltpu.SemaphoreType.DMA,
        ],
    )


def kernel(x, edge_index, batch, W1, a_src1, a_dst1, b1,
           W2, a_src2, a_dst2, b2, gate_W, gate_b, fin_W, fin_b):
    f32 = jnp.float32
    xp = jnp.pad(x, ((0, _NPAD - _N), (0, 0)))
    srcp = jnp.pad(edge_index[0], (0, _EPAD - _E)).reshape(_NW, _NCHUNK, _CPE)
    dstp = jnp.pad(edge_index[1], (0, _EPAD - _E)).reshape(_NW, _NCHUNK, _CPE)
    batch3 = jnp.pad(batch, (0, _NPAD - _N),
                     constant_values=_B).reshape(_NB, 1, _R)
    a21 = jnp.stack([a_src1, a_dst1], axis=1)
    a22 = jnp.stack([a_src2, a_dst2], axis=1)
    b1r = b1.reshape(1, _D)
    b2r = b2.reshape(1, _D)
    gf = jnp.concatenate([gate_W, fin_W], axis=1)
    gb2 = jnp.concatenate([gate_b, jnp.zeros((1,), f32)]).reshape(1, 2)
    fbr = fin_b.reshape(1, 1)

    sc_edge = _get_sc_edge()
    h1lo, h1hi, sa1, gm1 = _tc_stage1(xp, W1, a21)
    gv1 = jnp.full((16,), gm1[0, 0] + gm1[0, 1], dtype=f32)
    o1, d1 = sc_edge(srcp, dstp, sa1[:, 0], sa1[:, 1], gv1, h1lo, h1hi)

    h2lo, h2hi, sa2, gm2 = _tc_stage2(o1, d1[:, :, None], b1r, W2, a22)
    gv2 = jnp.full((16,), gm2[0, 0] + gm2[0, 1], dtype=f32)
    o2, d2 = sc_edge(srcp, dstp, sa2[:, 0], sa2[:, 1], gv2, h2lo, h2hi)

    gy, gm3 = _tc_final(o2, d2[:, :, None], b2r, gf, gb2)
    _, outf = _tc_pool(gy, batch3, gm3, fbr)
    return outf[:, 0]
